# baseline (speedup 1.0000x reference)
import jax
import jax.numpy as jnp
from jax.experimental import pallas as pl
from jax.experimental.pallas import tpu as pltpu


def _body(x_ref, w1_ref, b1_ref, w2_ref, b2_ref, o_ref, k):
    x = x_ref[0]
    H, W, C = x.shape
    Ho, Wo = H - k + 1, W - k + 1
    M = Ho * Wo
    taps = [x[ki:ki + Ho, kj:kj + Wo, :]
            for ki in range(k) for kj in range(k)]
    p = jnp.concatenate(taps, axis=-1).reshape(M, C * k * k)
    h = jnp.dot(p, w1_ref[...], preferred_element_type=jnp.float32)
    h = jnp.maximum(h + b1_ref[...], 0.0).astype(jnp.bfloat16)
    y = jax.lax.dot_general(w2_ref[...], h, (((1,), (1,)), ((), ())),
                            preferred_element_type=jnp.float32)
    y = y + b2_ref[...]
    o_ref[0] = jnp.maximum(y, 0.0).astype(o_ref.dtype)


def kernel(x, w1, b1, w2, b2):
    N, C, H, W = x.shape
    cmid, _, k, _ = w1.shape
    cout = w2.shape[0]
    Ho, Wo = H - k + 1, W - k + 1
    M = Ho * Wo

    xt = x.transpose(0, 2, 3, 1).astype(jnp.bfloat16)
    w1m = w1.transpose(2, 3, 1, 0).reshape(k * k * C, cmid).astype(jnp.bfloat16)
    w2m = w2.reshape(cout, cmid).astype(jnp.bfloat16)
    b1r = b1.reshape(1, cmid).astype(jnp.float32)
    b2c = b2.reshape(cout, 1).astype(jnp.float32)

    flops = 2 * N * M * (C * k * k) * cmid + 2 * N * M * cmid * cout
    bytes_accessed = (x.size * 4 + w1m.size * 2 + w2m.size * 2
                      + b1r.size * 4 + b2c.size * 4 + N * cout * M * 4)

    chunks = 4 if N % 4 == 0 else 1
    nc = N // chunks

    call = pl.pallas_call(
        lambda *refs: _body(*refs, k),
        out_shape=jax.ShapeDtypeStruct((nc, cout, M), jnp.bfloat16),
        grid=(nc,),
        in_specs=[
            pl.BlockSpec((1, H, W, C), lambda i: (i, 0, 0, 0)),
            pl.BlockSpec((k * k * C, cmid), lambda i: (0, 0)),
            pl.BlockSpec((1, cmid), lambda i: (0, 0)),
            pl.BlockSpec((cout, cmid), lambda i: (0, 0)),
            pl.BlockSpec((cout, 1), lambda i: (0, 0)),
        ],
        out_specs=pl.BlockSpec((1, cout, M), lambda i: (i, 0, 0)),
        compiler_params=pltpu.CompilerParams(
            dimension_semantics=("parallel",),
            vmem_limit_bytes=100 * 2 ** 20,
        ),
        cost_estimate=pl.CostEstimate(
            flops=flops // chunks, transcendentals=0,
            bytes_accessed=bytes_accessed // chunks),
    )

    outs = []
    for ci in range(chunks):
        oc = call(xt[ci * nc:(ci + 1) * nc], w1m, b1r, w2m, b2c)
        outs.append(oc.astype(jnp.float32).reshape(nc, cout, Ho, Wo))
    if chunks == 1:
        return outs[0]
    return jnp.concatenate(outs, axis=0)


# speedup vs baseline: 1.0833x; 1.0833x over previous
import jax
import jax.numpy as jnp
from jax.experimental import pallas as pl
from jax.experimental.pallas import tpu as pltpu


def _body(x_ref, w1_ref, b1_ref, w2_ref, b2_ref, o_ref, k):
    x = x_ref[0]
    H, Wp, C = x.shape
    Ho, Wo = H - k + 1, Wp - k + 1
    M = Ho * Wo
    taps = [x[ki:ki + Ho, kj:kj + Wo, :]
            for ki in range(k) for kj in range(k)]
    p = jnp.concatenate(taps, axis=-1).reshape(M, C * k * k)
    h = jnp.dot(p, w1_ref[...], preferred_element_type=jnp.float32)
    h = jnp.maximum(h + b1_ref[...], 0.0).astype(jnp.bfloat16)
    y = jax.lax.dot_general(w2_ref[...], h, (((1,), (1,)), ((), ())),
                            preferred_element_type=jnp.float32)
    y = y + b2_ref[...]
    o_ref[0] = jnp.maximum(y, 0.0).astype(o_ref.dtype)


def kernel(x, w1, b1, w2, b2):
    N, C, H, W = x.shape
    cmid, _, k, _ = w1.shape
    cout = w2.shape[0]
    Ho, Wo = H - k + 1, W - k + 1
    M = Ho * Wo

    xt = jnp.pad(x.transpose(0, 2, 3, 1),
                 ((0, 0), (0, 0), (0, k - 1), (0, 0))).astype(jnp.bfloat16)
    Wp = W + k - 1
    Mp = Ho * W
    w1m = w1.transpose(2, 3, 1, 0).reshape(k * k * C, cmid).astype(jnp.bfloat16)
    w2m = w2.reshape(cout, cmid).astype(jnp.bfloat16)
    b1r = b1.reshape(1, cmid).astype(jnp.float32)
    b2c = b2.reshape(cout, 1).astype(jnp.float32)

    flops = 2 * N * M * (C * k * k) * cmid + 2 * N * M * cmid * cout
    bytes_accessed = (x.size * 4 + w1m.size * 2 + w2m.size * 2
                      + b1r.size * 4 + b2c.size * 4 + N * cout * M * 4)

    chunks = 4 if N % 4 == 0 else 1
    nc = N // chunks

    out = pl.pallas_call(
        lambda *refs: _body(*refs, k),
        out_shape=jax.ShapeDtypeStruct((N, cout, Mp), jnp.bfloat16),
        grid=(N,),
        in_specs=[
            pl.BlockSpec((1, H, Wp, C), lambda i: (i, 0, 0, 0)),
            pl.BlockSpec((k * k * C, cmid), lambda i: (0, 0)),
            pl.BlockSpec((1, cmid), lambda i: (0, 0)),
            pl.BlockSpec((cout, cmid), lambda i: (0, 0)),
            pl.BlockSpec((cout, 1), lambda i: (0, 0)),
        ],
        out_specs=pl.BlockSpec((1, cout, Mp), lambda i: (i, 0, 0)),
        compiler_params=pltpu.CompilerParams(
            dimension_semantics=("parallel",),
            vmem_limit_bytes=100 * 2 ** 20,
        ),
        cost_estimate=pl.CostEstimate(
            flops=flops, transcendentals=0, bytes_accessed=bytes_accessed),
    )(xt, w1m, b1r, w2m, b2c)

    return out.reshape(N, cout, Ho, W)[:, :, :, :Wo].astype(jnp.float32)


# speedup vs baseline: 1.3030x; 1.2028x over previous
import jax
import jax.numpy as jnp
from jax.experimental import pallas as pl
from jax.experimental.pallas import tpu as pltpu


def _body(x_ref, w1_ref, b1_ref, w2_ref, b2_ref, o_ref, k):
    B, H, W, C = x_ref.shape
    Ho, Wo = H - k + 1, W - k + 1
    M = Ho * Wo
    for j in range(B):
        x = x_ref[j]
        taps = [x[ki:ki + Ho, kj:kj + Wo, :]
                for ki in range(k) for kj in range(k)]
        p = jnp.concatenate(taps, axis=-1).reshape(M, C * k * k)
        h = jnp.dot(p, w1_ref[...], preferred_element_type=jnp.float32)
        h = jnp.maximum(h + b1_ref[...], 0.0).astype(jnp.bfloat16)
        y = jax.lax.dot_general(w2_ref[...], h, (((1,), (1,)), ((), ())),
                                preferred_element_type=jnp.float32)
        y = y + b2_ref[...]
        o_ref[j] = jnp.maximum(y, 0.0).astype(o_ref.dtype)


def kernel(x, w1, b1, w2, b2):
    N, C, H, W = x.shape
    cmid, _, k, _ = w1.shape
    cout = w2.shape[0]
    Ho, Wo = H - k + 1, W - k + 1
    M = Ho * Wo

    xt = x.transpose(0, 2, 3, 1).astype(jnp.bfloat16)
    w1m = w1.transpose(2, 3, 1, 0).reshape(k * k * C, cmid).astype(jnp.bfloat16)
    w2m = w2.reshape(cout, cmid).astype(jnp.bfloat16)
    b1r = b1.reshape(1, cmid).astype(jnp.float32)
    b2c = b2.reshape(cout, 1).astype(jnp.float32)

    flops = 2 * N * M * (C * k * k) * cmid + 2 * N * M * cmid * cout
    bytes_accessed = (x.size * 4 + w1m.size * 2 + w2m.size * 2
                      + b1r.size * 4 + b2c.size * 4 + N * cout * M * 4)

    chunks = 4 if N % 4 == 0 else 1
    nc = N // chunks

    B = 4 if N % 4 == 0 else 1
    out = pl.pallas_call(
        lambda *refs: _body(*refs, k),
        out_shape=jax.ShapeDtypeStruct((N, cout, M), jnp.bfloat16),
        grid=(N // B,),
        in_specs=[
            pl.BlockSpec((B, H, W, C), lambda i: (i, 0, 0, 0)),
            pl.BlockSpec((k * k * C, cmid), lambda i: (0, 0)),
            pl.BlockSpec((1, cmid), lambda i: (0, 0)),
            pl.BlockSpec((cout, cmid), lambda i: (0, 0)),
            pl.BlockSpec((cout, 1), lambda i: (0, 0)),
        ],
        out_specs=pl.BlockSpec((B, cout, M), lambda i: (i, 0, 0)),
        compiler_params=pltpu.CompilerParams(
            dimension_semantics=("parallel",),
            vmem_limit_bytes=100 * 2 ** 20,
        ),
        cost_estimate=pl.CostEstimate(
            flops=flops, transcendentals=0, bytes_accessed=bytes_accessed),
    )(xt, w1m, b1r, w2m, b2c)

    return out.astype(jnp.float32).reshape(N, cout, Ho, Wo)
